# Initial kernel scaffold; baseline (speedup 1.0000x reference)
import jax
import jax.numpy as jnp
from jax.experimental import pallas as pl


def kernel(x, gate_weights, expert_weights, expert_min, out_w, out_b):
    raise NotImplementedError("write your pallas kernel here")


# speedup vs baseline: 5.3369x; 5.3369x over previous
import jax
import jax.numpy as jnp
from jax.experimental import pallas as pl


def _moe_body(h_ref, gw_ref, w_ref, em_ref, ow_ref, ob_ref,
              gates_ref, y_ref, emo_ref):
    h = h_ref[...]
    ow = ow_ref[...]
    b = ob_ref[0, 0]

    gates_ref[...] = jax.lax.dot_general(
        h, gw_ref[...], (((1,), (1,)), ((), ())),
        preferred_element_type=jnp.float32)

    v = jnp.sum(w_ref[...] * ow[None, :, :], axis=2)

    y_ref[...] = jax.lax.dot_general(
        h, v, (((1,), (1,)), ((), ())),
        preferred_element_type=jnp.float32) + b

    vmin = jax.lax.dot_general(
        em_ref[...], ow, (((1,), (1,)), ((), ())),
        preferred_element_type=jnp.float32)
    emo_ref[...] = jax.lax.dot_general(
        h, vmin, (((1,), (0,)), ((), ())),
        preferred_element_type=jnp.float32) + b


def kernel(x, gate_weights, expert_weights, expert_min, out_w, out_b):
    B, _, H = x.shape
    E = expert_weights.shape[0]
    h = x[:, 0, :]
    ob2 = out_b.reshape(1, 1)

    gates, y2, emo = pl.pallas_call(
        _moe_body,
        out_shape=[
            jax.ShapeDtypeStruct((B, E), jnp.float32),
            jax.ShapeDtypeStruct((B, E), jnp.float32),
            jax.ShapeDtypeStruct((B, 1), jnp.float32),
        ],
    )(h, gate_weights, expert_weights, expert_min, out_w, ob2)

    return (gates, y2.reshape(B, E, 1), emo)


# speedup vs baseline: 5.3694x; 1.0061x over previous
import functools

import jax
import jax.numpy as jnp
from jax.experimental import pallas as pl
from jax.experimental.pallas import tpu as pltpu

EB = 8


def _moe_body(h_ref, gw_ref, w_ref, em_ref, ow_ref, ob_ref,
              gates_ref, y_ref, emo_ref, v_scr, *, nb):
    i = pl.program_id(0)
    ow = ow_ref[...]

    v_scr[pl.ds(i * EB, EB), :] = jnp.sum(w_ref[...] * ow[None, :, :], axis=2)

    @pl.when(i == 0)
    def _():
        b = ob_ref[0, 0]
        h = h_ref[...]
        gates_ref[...] = jax.lax.dot_general(
            h, gw_ref[...], (((1,), (1,)), ((), ())),
            preferred_element_type=jnp.float32)
        vmin = jax.lax.dot_general(
            em_ref[...], ow, (((1,), (1,)), ((), ())),
            preferred_element_type=jnp.float32)
        emo_ref[...] = jax.lax.dot_general(
            h, vmin, (((1,), (0,)), ((), ()))) + b

    @pl.when(i == nb - 1)
    def _():
        y_ref[...] = jax.lax.dot_general(
            h_ref[...], v_scr[...], (((1,), (1,)), ((), ())),
            preferred_element_type=jnp.float32) + ob_ref[0, 0]


def kernel(x, gate_weights, expert_weights, expert_min, out_w, out_b):
    B, _, H = x.shape
    E, _, S = expert_weights.shape
    nb = E // EB
    h = x[:, 0, :]
    ob2 = out_b.reshape(1, 1)

    gates, y2, emo = pl.pallas_call(
        functools.partial(_moe_body, nb=nb),
        grid=(nb,),
        in_specs=[
            pl.BlockSpec((B, H), lambda i: (0, 0)),
            pl.BlockSpec((E, H), lambda i: (0, 0)),
            pl.BlockSpec((EB, H, S), lambda i: (i, 0, 0)),
            pl.BlockSpec((H, S), lambda i: (0, 0)),
            pl.BlockSpec((1, S), lambda i: (0, 0)),
            pl.BlockSpec((1, 1), lambda i: (0, 0)),
        ],
        out_specs=[
            pl.BlockSpec((B, E), lambda i: (0, 0)),
            pl.BlockSpec((B, E), lambda i: (0, 0)),
            pl.BlockSpec((B, 1), lambda i: (0, 0)),
        ],
        out_shape=[
            jax.ShapeDtypeStruct((B, E), jnp.float32),
            jax.ShapeDtypeStruct((B, E), jnp.float32),
            jax.ShapeDtypeStruct((B, 1), jnp.float32),
        ],
        scratch_shapes=[pltpu.VMEM((E, H), jnp.float32)],
    )(h, gate_weights, expert_weights, expert_min, out_w, ob2)

    return (gates, y2.reshape(B, E, 1), emo)


# speedup vs baseline: 5.4695x; 1.0186x over previous
import jax
import jax.numpy as jnp
from jax.experimental import pallas as pl
from jax.experimental.pallas import tpu as pltpu

NCHUNK = 8


def _moe_body(h_ref, gw_ref, w_hbm, em_ref, ow_ref, ob_ref,
              gates_ref, y_ref, emo_ref, w_vmem, sems):
    E = w_vmem.shape[0]
    ce = E // NCHUNK
    copies = [
        pltpu.make_async_copy(
            w_hbm.at[pl.ds(k * ce, ce)], w_vmem.at[pl.ds(k * ce, ce)],
            sems.at[k])
        for k in range(NCHUNK)
    ]
    for c in copies:
        c.start()

    ow = ow_ref[...]
    b = ob_ref[0, 0]
    h = h_ref[...]

    gates_ref[...] = jax.lax.dot_general(
        h, gw_ref[...], (((1,), (1,)), ((), ())),
        preferred_element_type=jnp.float32)

    vmin = jax.lax.dot_general(
        em_ref[...], ow, (((1,), (1,)), ((), ())),
        preferred_element_type=jnp.float32)
    emo_ref[...] = jax.lax.dot_general(
        h, vmin, (((1,), (0,)), ((), ()))) + b

    vparts = []
    for k, c in enumerate(copies):
        c.wait()
        vparts.append(
            jnp.sum(w_vmem[pl.ds(k * ce, ce)] * ow[None, :, :], axis=2))
    v = jnp.concatenate(vparts, axis=0)

    y_ref[...] = jax.lax.dot_general(
        h, v, (((1,), (1,)), ((), ())),
        preferred_element_type=jnp.float32) + b


def kernel(x, gate_weights, expert_weights, expert_min, out_w, out_b):
    B, _, H = x.shape
    E, _, S = expert_weights.shape
    h = x[:, 0, :]
    ob2 = out_b.reshape(1, 1)

    gates, y2, emo = pl.pallas_call(
        _moe_body,
        in_specs=[
            pl.BlockSpec(memory_space=pltpu.VMEM),
            pl.BlockSpec(memory_space=pltpu.VMEM),
            pl.BlockSpec(memory_space=pltpu.MemorySpace.HBM),
            pl.BlockSpec(memory_space=pltpu.VMEM),
            pl.BlockSpec(memory_space=pltpu.VMEM),
            pl.BlockSpec(memory_space=pltpu.VMEM),
        ],
        out_shape=[
            jax.ShapeDtypeStruct((B, E), jnp.float32),
            jax.ShapeDtypeStruct((B, E), jnp.float32),
            jax.ShapeDtypeStruct((B, 1), jnp.float32),
        ],
        scratch_shapes=[
            pltpu.VMEM((E, H, S), jnp.float32),
            pltpu.SemaphoreType.DMA((NCHUNK,)),
        ],
    )(h, gate_weights, expert_weights, expert_min, out_w, ob2)

    return (gates, y2.reshape(B, E, 1), emo)
